# baseline (speedup 1.0000x reference)
import functools

import jax
import jax.numpy as jnp
from jax import lax
from jax.experimental import pallas as pl
from jax.experimental.pallas import tpu as pltpu
from jax.experimental.pallas import tpu_sc as plsc

N = 10000
NPAD = 10240
D = 128
E = 320000
EPAD = 327680
ER = EPAD // 128
NC, NS = 2, 16
NW = NC * NS
TROWS = ER // NW
SROWS = NPAD // NS
GARB = N
F32 = jnp.float32
I32 = jnp.int32

@functools.lru_cache(maxsize=1)
def _mesh():
    return plsc.VectorSubcoreMesh(core_axis_name="c", subcore_axis_name="s",
                                  num_cores=NC, num_subcores=NS)


def _zero_1d(ref, nchunks):
    z = jnp.zeros((16,), ref.dtype)

    def body(i, _):
        ref[pl.ds(i * 16, 16)] = z
        return 0

    lax.fori_loop(0, nchunks, body, 0)


def _edge_body(src_h, dst_h, live_h, mask_h, liveo_h, srce_h, dste_h, deg_h,
               sidx, didx, lv, mg, md, lo, se, de, zc, mask_s,
               acc_s, sem):
    c = lax.axis_index("c")
    s = lax.axis_index("s")
    w = s * NC + c
    _zero_1d(zc, 8)
    pltpu.sync_copy(mask_h.at[pl.ds(s * SROWS, SROWS)],
                    mask_s.at[pl.ds(s * SROWS, SROWS)])
    for q in range(SROWS // 128):
        pltpu.sync_copy(zc, acc_s.at[pl.ds(s * SROWS + q * 128, 128)])
    plsc.subcore_barrier()

    base = w * TROWS
    iot = lax.iota(I32, 16)

    def outer(t, _):
        r0 = base + t * 8
        pltpu.sync_copy(src_h.at[pl.ds(r0, 8)], sidx)
        pltpu.sync_copy(dst_h.at[pl.ds(r0, 8)], didx)
        pltpu.sync_copy(live_h.at[pl.ds(r0, 8)], lv)
        cps = [pltpu.async_copy(mask_s.at[sidx.at[j]], mg.at[j], sem)
               for j in range(8)]
        cps += [pltpu.async_copy(mask_s.at[didx.at[j]], md.at[j], sem)
                for j in range(8)]
        for cp in cps:
            cp.wait()

        def comp(i, _):
            j = i // 8
            cc = i % 8
            sl = pl.ds(cc * 16, 16)
            mgv = mg[j, sl]
            lnew = lv[j, sl] * mgv * md[j, sl]
            keep = (lnew > 0.0) | (mgv == 0.0)
            garb = GARB + cc * 16 + iot
            se[j, sl] = jnp.where(keep, sidx[j, sl], garb)
            de[j, sl] = didx[j, sl]
            lo[j, sl] = lnew
            return 0

        lax.fori_loop(0, 64, comp, 0)
        pltpu.sync_copy(lo, liveo_h.at[pl.ds(r0, 8)])
        pltpu.sync_copy(se, srce_h.at[pl.ds(r0, 8)])
        pltpu.sync_copy(de, dste_h.at[pl.ds(r0, 8)])
        cps = [pltpu.async_copy(lo.at[j], acc_s.at[de.at[j]], sem, add=True)
               for j in range(8)]
        for cp in cps:
            cp.wait()
        return 0

    lax.fori_loop(0, TROWS // 8, outer, 0)
    plsc.subcore_barrier()
    pltpu.sync_copy(acc_s.at[pl.ds(s * SROWS, SROWS)],
                    deg_h.at[c, pl.ds(s * SROWS, SROWS)])


def _msg_body(hp_h, srce_h, dste_h, acc_h, sidx, didx, rows,
              acc_s, gsem, semA, semB):
    c = lax.axis_index("c")
    s = lax.axis_index("s")
    def zb(i, _):
        r = i // 8
        cc = i % 8
        rows[0, r, pl.ds(cc * 16, 16)] = jnp.zeros((16,), F32)
        return 0

    lax.fori_loop(0, 1024, zb, 0)
    for q in range(SROWS // 128):
        pltpu.sync_copy(rows.at[0], acc_s.at[pl.ds(s * SROWS + q * 128, 128)])
    plsc.subcore_barrier()

    base = (s * NC + c) * TROWS

    def outer(t, _):
        bank = jnp.bitwise_and(t, 1)
        r0 = base + t * 8
        pltpu.sync_copy(srce_h.at[pl.ds(r0, 8)], sidx.at[bank])
        pltpu.sync_copy(dste_h.at[pl.ds(r0, 8)], didx.at[bank])
        for j in range(8):
            b = j & 1
            sem = semA if b == 0 else semB
            if j >= 2:
                pltpu.make_async_copy(rows.at[b],
                                      acc_s.at[didx.at[bank, j]],
                                      sem).wait()
            else:
                @pl.when(t > 0)
                def _():
                    pltpu.make_async_copy(rows.at[b],
                                          acc_s.at[didx.at[bank, j]],
                                          sem).wait()
            pltpu.async_copy(hp_h.at[sidx.at[bank, j]], rows.at[b],
                             gsem).wait()
            pltpu.async_copy(rows.at[b], acc_s.at[didx.at[bank, j]], sem,
                             add=True)
        return 0

    lax.fori_loop(0, TROWS // 8, outer, 0)
    pltpu.make_async_copy(rows.at[0], acc_s.at[didx.at[0, 0]], semA).wait()
    pltpu.make_async_copy(rows.at[1], acc_s.at[didx.at[0, 1]], semB).wait()
    plsc.subcore_barrier()
    pltpu.sync_copy(acc_s.at[pl.ds(s * SROWS, SROWS)],
                    acc_h.at[c, pl.ds(s * SROWS, SROWS)])


def _score_body(t_h, srce_h, dste_h, sacc_h, sidx, didx, tg, zc, t_s,
                acc_s, sem):
    c = lax.axis_index("c")
    s = lax.axis_index("s")
    w = s * NC + c
    _zero_1d(zc, 8)
    pltpu.sync_copy(t_h.at[pl.ds(s * SROWS, SROWS)],
                    t_s.at[pl.ds(s * SROWS, SROWS)])
    for q in range(SROWS // 128):
        pltpu.sync_copy(zc, acc_s.at[pl.ds(s * SROWS + q * 128, 128)])
    plsc.subcore_barrier()

    base = w * TROWS

    def outer(t, _):
        r0 = base + t * 8
        pltpu.sync_copy(srce_h.at[pl.ds(r0, 8)], sidx)
        pltpu.sync_copy(dste_h.at[pl.ds(r0, 8)], didx)
        cps = [pltpu.async_copy(t_s.at[sidx.at[j]], tg.at[j], sem)
               for j in range(8)]
        for cp in cps:
            cp.wait()
        cps = [pltpu.async_copy(tg.at[j], acc_s.at[didx.at[j]], sem,
                                add=True)
               for j in range(8)]
        for cp in cps:
            cp.wait()
        return 0

    lax.fori_loop(0, TROWS // 8, outer, 0)
    plsc.subcore_barrier()
    pltpu.sync_copy(acc_s.at[pl.ds(s * SROWS, SROWS)],
                    sacc_h.at[c, pl.ds(s * SROWS, SROWS)])


@functools.lru_cache(maxsize=1)
def _sc_kernels():
    mesh = _mesh()
    edge = pl.kernel(
        _edge_body,
        out_type=[
            jax.ShapeDtypeStruct((ER, 128), F32),
            jax.ShapeDtypeStruct((ER, 128), I32),
            jax.ShapeDtypeStruct((ER, 128), I32),
            jax.ShapeDtypeStruct((NC, NPAD), F32),
        ],
        mesh=mesh,
        scratch_types=[
            pltpu.VMEM((8, 128), I32),
            pltpu.VMEM((8, 128), I32),
            pltpu.VMEM((8, 128), F32),
            pltpu.VMEM((8, 128), F32),
            pltpu.VMEM((8, 128), F32),
            pltpu.VMEM((8, 128), F32),
            pltpu.VMEM((8, 128), I32),
            pltpu.VMEM((8, 128), I32),
            pltpu.VMEM((128,), F32),
            pltpu.VMEM_SHARED((NPAD,), F32),
            pltpu.VMEM_SHARED((NPAD,), F32),
            pltpu.SemaphoreType.DMA,
        ],
    )
    msg = pl.kernel(
        _msg_body,
        out_type=jax.ShapeDtypeStruct((NC, NPAD, 128), F32),
        mesh=mesh,
        scratch_types=[
            pltpu.VMEM((2, 8, 128), I32),
            pltpu.VMEM((2, 8, 128), I32),
            pltpu.VMEM((2, 128, 128), F32),
            pltpu.VMEM_SHARED((NPAD, 128), F32),
            pltpu.SemaphoreType.DMA,
            pltpu.SemaphoreType.DMA,
            pltpu.SemaphoreType.DMA,
        ],
    )
    score = pl.kernel(
        _score_body,
        out_type=jax.ShapeDtypeStruct((NC, NPAD), F32),
        mesh=mesh,
        scratch_types=[
            pltpu.VMEM((8, 128), I32),
            pltpu.VMEM((8, 128), I32),
            pltpu.VMEM((8, 128), F32),
            pltpu.VMEM((128,), F32),
            pltpu.VMEM_SHARED((NPAD,), F32),
            pltpu.VMEM_SHARED((NPAD,), F32),
            pltpu.SemaphoreType.DMA,
        ],
    )
    return edge, msg, score


def _edge_kernel(*args):
    return _sc_kernels()[0](*args)


def _msg_kernel(*args):
    return _sc_kernels()[1](*args)


def _score_kernel(*args):
    return _sc_kernels()[2](*args)


def _tc_prep_body(x_ref, w_ref, deg_ref, m_ref, hp_ref, dinv_ref):
    deg = deg_ref[0] + deg_ref[1] + m_ref[...]
    dinv = jnp.where(deg > 0, lax.rsqrt(deg), 0.0)
    h = jnp.dot(x_ref[...], w_ref[...], preferred_element_type=F32)
    hp_ref[...] = h * dinv
    dinv_ref[...] = dinv


_tc_prep = pl.pallas_call(
    _tc_prep_body,
    out_shape=[jax.ShapeDtypeStruct((NPAD, 128), F32),
               jax.ShapeDtypeStruct((NPAD, 1), F32)],
)


def _tc_post_body(acc_ref, hp_ref, dinv_ref, b_ref, m_ref, wrel_ref,
                  wroot_ref, g_ref, t_ref, r_ref):
    tot = (acc_ref[0] + acc_ref[1] + hp_ref[...]) * dinv_ref[...]
    g = jnp.maximum(tot + b_ref[...], 0.0) * m_ref[...]
    g_ref[...] = g
    t_ref[...] = jnp.dot(g, wrel_ref[...], preferred_element_type=F32)
    r_ref[...] = jnp.dot(g, wroot_ref[...], preferred_element_type=F32)


_tc_post = pl.pallas_call(
    _tc_post_body,
    out_shape=[jax.ShapeDtypeStruct((NPAD, 128), F32),
               jax.ShapeDtypeStruct((NPAD, 1), F32),
               jax.ShapeDtypeStruct((NPAD, 1), F32)],
)


def _tc_pool_body(sacc_ref, r_ref, brel_ref, m_ref, g_ref,
                  mnew_ref, xnew_ref, ro_ref, *, k):
    score = sacc_ref[0] + sacc_ref[1] + r_ref[...] + brel_ref[0, 0]
    neg = jnp.float32(-jnp.inf)
    sm = jnp.where(m_ref[...] > 0, score, neg)
    bits = lax.bitcast_convert_type(sm, I32)
    key = jnp.where(bits < 0, bits ^ jnp.int32(0x7FFFFFFF), bits)

    def bis(i, p):
        cand = p + jnp.left_shift(jnp.int32(1), 30 - i)
        cnt = jnp.sum((key >= cand).astype(F32))
        return jnp.where(cnt >= k, cand, p)

    p = lax.fori_loop(0, 31, bis, jnp.int32(-2147483648))
    tie = key == p
    cgt = jnp.sum((key > p).astype(F32))
    need = jnp.float32(k) - cgt
    idx = lax.broadcasted_iota(I32, (NPAD, 1), 0)

    def bis2(i, lo):
        cand = lo + jnp.left_shift(jnp.int32(1), 13 - i)
        cnt = jnp.sum((tie & (idx < cand)).astype(F32))
        return jnp.where(cnt < need, cand, lo)

    lo = lax.fori_loop(0, 14, bis2, jnp.int32(0))
    sel = (key > p) | (tie & (idx <= lo))
    mnew = sel.astype(F32)
    mnew_ref[...] = mnew
    xn = g_ref[...] * jnp.tanh(sm) * mnew
    xnew_ref[...] = xn
    mean = jnp.sum(xn, axis=0, keepdims=True) * jnp.float32(1.0 / k)
    mx = jnp.max(jnp.where(mnew > 0, xn, neg), axis=0, keepdims=True)
    ro_ref[...] = jnp.concatenate([mean, mx], axis=1)


def _make_pool(k):
    return pl.pallas_call(
        functools.partial(_tc_pool_body, k=k),
        out_shape=[jax.ShapeDtypeStruct((NPAD, 1), F32),
                   jax.ShapeDtypeStruct((NPAD, 128), F32),
                   jax.ShapeDtypeStruct((1, 256), F32)],
    )


def _tc_head_body(r1, r2, r3, m1, bm1, m2, bm2, m3, bm3, y_ref):
    r = r1[...] + r2[...] + r3[...]
    h = jnp.maximum(jnp.dot(r, m1[...], preferred_element_type=F32)
                    + bm1[...], 0.0)
    h = jnp.maximum(jnp.dot(h, m2[...], preferred_element_type=F32)
                    + bm2[...], 0.0)
    y = jnp.dot(h, m3[...], preferred_element_type=F32) + bm3[...]
    mx = jnp.max(y, axis=1, keepdims=True)
    e = jnp.exp(y - mx)
    y_ref[...] = y - mx - jnp.log(jnp.sum(e, axis=1, keepdims=True))


_tc_head = pl.pallas_call(
    _tc_head_body, out_shape=jax.ShapeDtypeStruct((1, 10), F32))


def kernel(x, edge_index, W1, b1, Wrel1, brel1, Wroot1, W2, b2, Wrel2, brel2,
           Wroot2, W3, b3, Wrel3, brel3, Wroot3, M1, bm1, M2, bm2, M3, bm3):
    src = edge_index[0].astype(I32)
    dst = edge_index[1].astype(I32)
    padv = GARB + (jnp.arange(EPAD - E, dtype=I32) % 128)
    src2 = jnp.concatenate([src, padv]).reshape(ER, 128)
    dst2 = jnp.concatenate([dst, padv]).reshape(ER, 128)
    live = jnp.pad(jnp.ones((E,), F32), (0, EPAD - E)).reshape(ER, 128)
    X = jnp.pad(x.astype(F32), ((0, NPAD - N), (0, 0)))
    m = jnp.pad(jnp.ones((N, 1), F32), ((0, NPAD - N), (0, 0)))

    stages = [
        (W1, b1, Wrel1, brel1, Wroot1, 5000),
        (W2, b2, Wrel2, brel2, Wroot2, 2500),
        (W3, b3, Wrel3, brel3, Wroot3, 1250),
    ]
    ros = []
    for Wm, bv, Wr, br, Wt, k in stages:
        live, srce, dste, deg = _edge_kernel(src2, dst2, live,
                                             m.reshape(NPAD))
        hp, dinv = _tc_prep(X, Wm, deg.reshape(NC, NPAD, 1), m)
        acc = _msg_kernel(hp, srce, dste)
        g, t, r = _tc_post(acc, hp, dinv, bv.reshape(1, 128), m, Wr, Wt)
        sacc = _score_kernel(t.reshape(NPAD), srce, dste)
        m, X, ro = _make_pool(k)(sacc.reshape(NC, NPAD, 1), r,
                                 br.reshape(1, 1), m, g)
        ros.append(ro)

    return _tc_head(ros[0], ros[1], ros[2], M1, bm1.reshape(1, 128),
                    M2, bm2.reshape(1, 64), M3, bm3.reshape(1, 10))


# speedup vs baseline: 1.0744x; 1.0744x over previous
import functools

import jax
import jax.numpy as jnp
from jax import lax
from jax.experimental import pallas as pl
from jax.experimental.pallas import tpu as pltpu
from jax.experimental.pallas import tpu_sc as plsc

N = 10000
NPAD = 10240
D = 128
E = 320000
EPAD = 327680
ER = EPAD // 128
NC, NS = 2, 16
NW = NC * NS
TROWS = ER // NW
SROWS = NPAD // NS
GARB = N
F32 = jnp.float32
I32 = jnp.int32

@functools.lru_cache(maxsize=1)
def _mesh():
    return plsc.VectorSubcoreMesh(core_axis_name="c", subcore_axis_name="s",
                                  num_cores=NC, num_subcores=NS)


def _zero_1d(ref, nchunks):
    z = jnp.zeros((16,), ref.dtype)

    def body(i, _):
        ref[pl.ds(i * 16, 16)] = z
        return 0

    lax.fori_loop(0, nchunks, body, 0)


def _edge_body(src_h, dst_h, live_h, mask_h, liveo_h, srce_h, dste_h, deg_h,
               sidx, didx, lv, mg, md, lo, se, de, zc, mask_s,
               acc_s, sem):
    c = lax.axis_index("c")
    s = lax.axis_index("s")
    w = s * NC + c
    _zero_1d(zc, 8)
    pltpu.sync_copy(mask_h.at[pl.ds(s * SROWS, SROWS)],
                    mask_s.at[pl.ds(s * SROWS, SROWS)])
    for q in range(SROWS // 128):
        pltpu.sync_copy(zc, acc_s.at[pl.ds(s * SROWS + q * 128, 128)])
    plsc.subcore_barrier()

    base = w * TROWS
    iot = lax.iota(I32, 16)

    def outer(t, _):
        r0 = base + t * 8
        pltpu.sync_copy(src_h.at[pl.ds(r0, 8)], sidx)
        pltpu.sync_copy(dst_h.at[pl.ds(r0, 8)], didx)
        pltpu.sync_copy(live_h.at[pl.ds(r0, 8)], lv)
        cps = [pltpu.async_copy(mask_s.at[sidx.at[j]], mg.at[j], sem)
               for j in range(8)]
        cps += [pltpu.async_copy(mask_s.at[didx.at[j]], md.at[j], sem)
                for j in range(8)]
        for cp in cps:
            cp.wait()

        def comp(i, _):
            j = i // 8
            cc = i % 8
            sl = pl.ds(cc * 16, 16)
            mgv = mg[j, sl]
            lnew = lv[j, sl] * mgv * md[j, sl]
            keep = (lnew > 0.0) | (mgv == 0.0)
            garb = GARB + cc * 16 + iot
            se[j, sl] = jnp.where(keep, sidx[j, sl], garb)
            de[j, sl] = didx[j, sl]
            lo[j, sl] = lnew
            return 0

        lax.fori_loop(0, 64, comp, 0)
        pltpu.sync_copy(lo, liveo_h.at[pl.ds(r0, 8)])
        pltpu.sync_copy(se, srce_h.at[pl.ds(r0, 8)])
        pltpu.sync_copy(de, dste_h.at[pl.ds(r0, 8)])
        cps = [pltpu.async_copy(lo.at[j], acc_s.at[de.at[j]], sem, add=True)
               for j in range(8)]
        for cp in cps:
            cp.wait()
        return 0

    lax.fori_loop(0, TROWS // 8, outer, 0)
    plsc.subcore_barrier()
    pltpu.sync_copy(acc_s.at[pl.ds(s * SROWS, SROWS)],
                    deg_h.at[c, pl.ds(s * SROWS, SROWS)])


def _msg_body(hp_h, srce_h, dste_h, acc_h, sidx, didx, rows,
              acc_s, gsemA, gsemB, ssemA, ssemB):
    c = lax.axis_index("c")
    s = lax.axis_index("s")
    def zb(i, _):
        r = i // 8
        cc = i % 8
        rows[0, r, pl.ds(cc * 16, 16)] = jnp.zeros((16,), F32)
        return 0

    lax.fori_loop(0, 1024, zb, 0)
    for q in range(SROWS // 128):
        pltpu.sync_copy(rows.at[0], acc_s.at[pl.ds(s * SROWS + q * 128, 128)])
    plsc.subcore_barrier()

    base = (s * NC + c) * TROWS
    nchunks = TROWS // 8
    gsems = (gsemA, gsemB)
    ssems = (ssemA, ssemB)

    def wait_s(b):
        pltpu.make_async_copy(rows.at[1 - b], acc_s.at[didx.at[0, 0]],
                              ssems[1 - b]).wait()

    def wait_g(b):
        pltpu.make_async_copy(hp_h.at[sidx.at[0, 0]], rows.at[b],
                              gsems[b]).wait()

    pltpu.sync_copy(srce_h.at[pl.ds(base, 8)], sidx.at[0])
    pltpu.sync_copy(dste_h.at[pl.ds(base, 8)], didx.at[0])
    pltpu.async_copy(hp_h.at[sidx.at[0, 0]], rows.at[0], gsemA)

    def outer(t, _):
        bank = lax.rem(t, 3)
        nbank = lax.rem(t + 1, 3)

        @pl.when(t < nchunks - 1)
        def _():
            r0 = base + (t + 1) * 8
            pltpu.sync_copy(srce_h.at[pl.ds(r0, 8)], sidx.at[nbank])
            pltpu.sync_copy(dste_h.at[pl.ds(r0, 8)], didx.at[nbank])

        for j in range(8):
            b = j & 1
            if j > 0:
                wait_s(b)
            else:
                @pl.when(t > 0)
                def _():
                    wait_s(b)
            if j < 7:
                pltpu.async_copy(hp_h.at[sidx.at[bank, j + 1]],
                                 rows.at[1 - b], gsems[1 - b])
            else:
                @pl.when(t < nchunks - 1)
                def _():
                    pltpu.async_copy(hp_h.at[sidx.at[nbank, 0]],
                                     rows.at[1 - b], gsems[1 - b])
            wait_g(b)
            pltpu.async_copy(rows.at[b], acc_s.at[didx.at[bank, j]],
                             ssems[b], add=True)
        return 0

    lax.fori_loop(0, nchunks, outer, 0)
    pltpu.make_async_copy(rows.at[1], acc_s.at[didx.at[0, 0]], ssemB).wait()
    plsc.subcore_barrier()
    pltpu.sync_copy(acc_s.at[pl.ds(s * SROWS, SROWS)],
                    acc_h.at[c, pl.ds(s * SROWS, SROWS)])


def _score_body(t_h, srce_h, dste_h, sacc_h, sidx, didx, tg, zc, t_s,
                acc_s, sem):
    c = lax.axis_index("c")
    s = lax.axis_index("s")
    w = s * NC + c
    _zero_1d(zc, 8)
    pltpu.sync_copy(t_h.at[pl.ds(s * SROWS, SROWS)],
                    t_s.at[pl.ds(s * SROWS, SROWS)])
    for q in range(SROWS // 128):
        pltpu.sync_copy(zc, acc_s.at[pl.ds(s * SROWS + q * 128, 128)])
    plsc.subcore_barrier()

    base = w * TROWS

    def outer(t, _):
        r0 = base + t * 8
        pltpu.sync_copy(srce_h.at[pl.ds(r0, 8)], sidx)
        pltpu.sync_copy(dste_h.at[pl.ds(r0, 8)], didx)
        cps = [pltpu.async_copy(t_s.at[sidx.at[j]], tg.at[j], sem)
               for j in range(8)]
        for cp in cps:
            cp.wait()
        cps = [pltpu.async_copy(tg.at[j], acc_s.at[didx.at[j]], sem,
                                add=True)
               for j in range(8)]
        for cp in cps:
            cp.wait()
        return 0

    lax.fori_loop(0, TROWS // 8, outer, 0)
    plsc.subcore_barrier()
    pltpu.sync_copy(acc_s.at[pl.ds(s * SROWS, SROWS)],
                    sacc_h.at[c, pl.ds(s * SROWS, SROWS)])


@functools.lru_cache(maxsize=1)
def _sc_kernels():
    mesh = _mesh()
    edge = pl.kernel(
        _edge_body,
        out_type=[
            jax.ShapeDtypeStruct((ER, 128), F32),
            jax.ShapeDtypeStruct((ER, 128), I32),
            jax.ShapeDtypeStruct((ER, 128), I32),
            jax.ShapeDtypeStruct((NC, NPAD), F32),
        ],
        mesh=mesh,
        scratch_types=[
            pltpu.VMEM((8, 128), I32),
            pltpu.VMEM((8, 128), I32),
            pltpu.VMEM((8, 128), F32),
            pltpu.VMEM((8, 128), F32),
            pltpu.VMEM((8, 128), F32),
            pltpu.VMEM((8, 128), F32),
            pltpu.VMEM((8, 128), I32),
            pltpu.VMEM((8, 128), I32),
            pltpu.VMEM((128,), F32),
            pltpu.VMEM_SHARED((NPAD,), F32),
            pltpu.VMEM_SHARED((NPAD,), F32),
            pltpu.SemaphoreType.DMA,
        ],
    )
    msg = pl.kernel(
        _msg_body,
        out_type=jax.ShapeDtypeStruct((NC, NPAD, 128), F32),
        mesh=mesh,
        scratch_types=[
            pltpu.VMEM((3, 8, 128), I32),
            pltpu.VMEM((3, 8, 128), I32),
            pltpu.VMEM((2, 128, 128), F32),
            pltpu.VMEM_SHARED((NPAD, 128), F32),
            pltpu.SemaphoreType.DMA,
            pltpu.SemaphoreType.DMA,
            pltpu.SemaphoreType.DMA,
            pltpu.SemaphoreType.DMA,
        ],
    )
    score = pl.kernel(
        _score_body,
        out_type=jax.ShapeDtypeStruct((NC, NPAD), F32),
        mesh=mesh,
        scratch_types=[
            pltpu.VMEM((8, 128), I32),
            pltpu.VMEM((8, 128), I32),
            pltpu.VMEM((8, 128), F32),
            pltpu.VMEM((128,), F32),
            pltpu.VMEM_SHARED((NPAD,), F32),
            pltpu.VMEM_SHARED((NPAD,), F32),
            pltpu.SemaphoreType.DMA,
        ],
    )
    return edge, msg, score


def _edge_kernel(*args):
    return _sc_kernels()[0](*args)


def _msg_kernel(*args):
    return _sc_kernels()[1](*args)


def _score_kernel(*args):
    return _sc_kernels()[2](*args)


def _tc_prep_body(x_ref, w_ref, deg_ref, m_ref, hp_ref, dinv_ref):
    deg = deg_ref[0] + deg_ref[1] + m_ref[...]
    dinv = jnp.where(deg > 0, lax.rsqrt(deg), 0.0)
    h = jnp.dot(x_ref[...], w_ref[...], preferred_element_type=F32)
    hp_ref[...] = h * dinv
    dinv_ref[...] = dinv


_tc_prep = pl.pallas_call(
    _tc_prep_body,
    out_shape=[jax.ShapeDtypeStruct((NPAD, 128), F32),
               jax.ShapeDtypeStruct((NPAD, 1), F32)],
)


def _tc_post_body(acc_ref, hp_ref, dinv_ref, b_ref, m_ref, wrel_ref,
                  wroot_ref, g_ref, t_ref, r_ref):
    tot = (acc_ref[0] + acc_ref[1] + hp_ref[...]) * dinv_ref[...]
    g = jnp.maximum(tot + b_ref[...], 0.0) * m_ref[...]
    g_ref[...] = g
    t_ref[...] = jnp.dot(g, wrel_ref[...], preferred_element_type=F32)
    r_ref[...] = jnp.dot(g, wroot_ref[...], preferred_element_type=F32)


_tc_post = pl.pallas_call(
    _tc_post_body,
    out_shape=[jax.ShapeDtypeStruct((NPAD, 128), F32),
               jax.ShapeDtypeStruct((NPAD, 1), F32),
               jax.ShapeDtypeStruct((NPAD, 1), F32)],
)


def _tc_pool_body(sacc_ref, r_ref, brel_ref, m_ref, g_ref,
                  mnew_ref, xnew_ref, ro_ref, *, k):
    score = sacc_ref[0] + sacc_ref[1] + r_ref[...] + brel_ref[0, 0]
    neg = jnp.float32(-jnp.inf)
    sm = jnp.where(m_ref[...] > 0, score, neg)
    bits = lax.bitcast_convert_type(sm, I32)
    key = jnp.where(bits < 0, bits ^ jnp.int32(0x7FFFFFFF), bits)

    def bis(i, p):
        cand = p + jnp.left_shift(jnp.int32(1), 30 - i)
        cnt = jnp.sum((key >= cand).astype(F32))
        return jnp.where(cnt >= k, cand, p)

    p = lax.fori_loop(0, 31, bis, jnp.int32(-2147483648))
    tie = key == p
    cgt = jnp.sum((key > p).astype(F32))
    need = jnp.float32(k) - cgt
    idx = lax.broadcasted_iota(I32, (NPAD, 1), 0)

    def bis2(i, lo):
        cand = lo + jnp.left_shift(jnp.int32(1), 13 - i)
        cnt = jnp.sum((tie & (idx < cand)).astype(F32))
        return jnp.where(cnt < need, cand, lo)

    lo = lax.fori_loop(0, 14, bis2, jnp.int32(0))
    sel = (key > p) | (tie & (idx <= lo))
    mnew = sel.astype(F32)
    mnew_ref[...] = mnew
    xn = g_ref[...] * jnp.tanh(sm) * mnew
    xnew_ref[...] = xn
    mean = jnp.sum(xn, axis=0, keepdims=True) * jnp.float32(1.0 / k)
    mx = jnp.max(jnp.where(mnew > 0, xn, neg), axis=0, keepdims=True)
    ro_ref[...] = jnp.concatenate([mean, mx], axis=1)


def _make_pool(k):
    return pl.pallas_call(
        functools.partial(_tc_pool_body, k=k),
        out_shape=[jax.ShapeDtypeStruct((NPAD, 1), F32),
                   jax.ShapeDtypeStruct((NPAD, 128), F32),
                   jax.ShapeDtypeStruct((1, 256), F32)],
    )


def _tc_head_body(r1, r2, r3, m1, bm1, m2, bm2, m3, bm3, y_ref):
    r = r1[...] + r2[...] + r3[...]
    h = jnp.maximum(jnp.dot(r, m1[...], preferred_element_type=F32)
                    + bm1[...], 0.0)
    h = jnp.maximum(jnp.dot(h, m2[...], preferred_element_type=F32)
                    + bm2[...], 0.0)
    y = jnp.dot(h, m3[...], preferred_element_type=F32) + bm3[...]
    mx = jnp.max(y, axis=1, keepdims=True)
    e = jnp.exp(y - mx)
    y_ref[...] = y - mx - jnp.log(jnp.sum(e, axis=1, keepdims=True))


_tc_head = pl.pallas_call(
    _tc_head_body, out_shape=jax.ShapeDtypeStruct((1, 10), F32))


def kernel(x, edge_index, W1, b1, Wrel1, brel1, Wroot1, W2, b2, Wrel2, brel2,
           Wroot2, W3, b3, Wrel3, brel3, Wroot3, M1, bm1, M2, bm2, M3, bm3):
    src = edge_index[0].astype(I32)
    dst = edge_index[1].astype(I32)
    padv = GARB + (jnp.arange(EPAD - E, dtype=I32) % 128)
    src2 = jnp.concatenate([src, padv]).reshape(ER, 128)
    dst2 = jnp.concatenate([dst, padv]).reshape(ER, 128)
    live = jnp.pad(jnp.ones((E,), F32), (0, EPAD - E)).reshape(ER, 128)
    X = jnp.pad(x.astype(F32), ((0, NPAD - N), (0, 0)))
    m = jnp.pad(jnp.ones((N, 1), F32), ((0, NPAD - N), (0, 0)))

    stages = [
        (W1, b1, Wrel1, brel1, Wroot1, 5000),
        (W2, b2, Wrel2, brel2, Wroot2, 2500),
        (W3, b3, Wrel3, brel3, Wroot3, 1250),
    ]
    ros = []
    for Wm, bv, Wr, br, Wt, k in stages:
        live, srce, dste, deg = _edge_kernel(src2, dst2, live,
                                             m.reshape(NPAD))
        hp, dinv = _tc_prep(X, Wm, deg.reshape(NC, NPAD, 1), m)
        acc = _msg_kernel(hp, srce, dste)
        g, t, r = _tc_post(acc, hp, dinv, bv.reshape(1, 128), m, Wr, Wt)
        sacc = _score_kernel(t.reshape(NPAD), srce, dste)
        m, X, ro = _make_pool(k)(sacc.reshape(NC, NPAD, 1), r,
                                 br.reshape(1, 1), m, g)
        ros.append(ro)

    return _tc_head(ros[0], ros[1], ros[2], M1, bm1.reshape(1, 128),
                    M2, bm2.reshape(1, 64), M3, bm3.reshape(1, 10))
